# baseline (speedup 1.0000x reference)
import jax
import jax.numpy as jnp
from jax import lax
from jax.experimental import pallas as pl
from jax.experimental.pallas import tpu as pltpu
from jax.experimental.pallas import tpu_sc as plsc

_B = 16384
_V = 1000000
_D = 64
_F = 128
_NC = 2
_NS = 16
_NW = _NC * _NS
_NIDX = 2 * _B
_PER_W = _NIDX // _NW
_RND = 2
_PER_RND = _PER_W // _RND
_NQ = _PER_RND // 128
_KR = _NQ * _D

_mesh = plsc.VectorSubcoreMesh(
    core_axis_name="c", subcore_axis_name="s",
    num_cores=_NC, num_subcores=_NS,
)


def _gather_body(tbl_hbm, addr_hbm, out_hbm, idx_v, rows_v, sem):
    wid = lax.axis_index("s") * _NC + lax.axis_index("c")
    for r in range(_RND):
        pltpu.sync_copy(addr_hbm.at[wid, r], idx_v)

        def _fire(k, _):
            pltpu.async_copy(tbl_hbm.at[idx_v.at[k]], rows_v.at[k], sem)
            return ()

        lax.fori_loop(0, _KR, _fire, ())
        pltpu.make_async_copy(out_hbm.at[wid, r], rows_v, sem).wait()
        pltpu.sync_copy(rows_v, out_hbm.at[wid, r])


_gather = pl.kernel(
    _gather_body,
    out_type=jax.ShapeDtypeStruct((_NW, _RND, _KR, 128), jnp.float32),
    mesh=_mesh,
    scratch_types=[
        pltpu.VMEM((_KR, 128), jnp.int32),
        pltpu.VMEM((_KR, 128), jnp.float32),
        pltpu.SemaphoreType.DMA,
    ],
)


def _mlp_body(top_ref, bot_ref, w1a_ref, w1b_ref, b1_ref, w2t_ref, b2_ref, out_ref):
    w1a = w1a_ref[...]
    w1b = w1b_ref[...]
    b1 = b1_ref[...]
    w2t = w2t_ref[...]
    b2 = b2_ref[...]
    for r in range(_RND):
        for q in range(_NQ):
            top = top_ref[0, r, q]
            bot = bot_ref[0, r, q]
            cond = (
                jnp.dot(w1a, top, preferred_element_type=jnp.float32)
                + jnp.dot(w1b, bot, preferred_element_type=jnp.float32)
                + b1
            )
            h = jnp.tanh(cond)
            score = lax.dot_general(
                h, w2t, (((0,), (0,)), ((), ())),
                preferred_element_type=jnp.float32,
            )
            out_ref[pl.ds((r * _NQ + q) * 128, 128), :] = score + b2


def _mlp(g5, w1a, w1b, b1c, w2t, b2r):
    return pl.pallas_call(
        _mlp_body,
        grid=(_NW // 2,),
        in_specs=[
            pl.BlockSpec((1, _RND, _NQ, _D, 128), lambda n: (n, 0, 0, 0, 0)),
            pl.BlockSpec((1, _RND, _NQ, _D, 128), lambda n: (n + _NW // 2, 0, 0, 0, 0)),
            pl.BlockSpec((_F, _D), lambda n: (0, 0)),
            pl.BlockSpec((_F, _D), lambda n: (0, 0)),
            pl.BlockSpec((_F, 1), lambda n: (0, 0)),
            pl.BlockSpec((_F, 2), lambda n: (0, 0)),
            pl.BlockSpec((1, 2), lambda n: (0, 0)),
        ],
        out_specs=pl.BlockSpec((_PER_W, 2), lambda n: (n, 0)),
        out_shape=jax.ShapeDtypeStruct((_B, 2), jnp.float32),
    )(g5, g5, w1a, w1b, b1c, w2t, b2r)


def kernel(x, x_mask, ents, batch_spos, batch_tpos, batch_sent_chars, emb, W1, b1, W2, b2):
    ents32 = ents.astype(jnp.int32)
    c = jnp.arange(_D, dtype=jnp.int32) * _V
    et = ents32.T.reshape(_NW, _RND, _NQ, 1, 128)
    addr = c.reshape(1, 1, 1, _D, 1) + et
    addr = addr.reshape(_NW, _RND, _KR, 128)

    tblf = emb.T.reshape(_V * _D)
    g = _gather(tblf, addr)
    g5 = g.reshape(_NW, _RND, _NQ, _D, 128)

    w1a = W1[:, :_D]
    w1b = W1[:, _D:]
    return _mlp(g5, w1a, w1b, b1.reshape(_F, 1), W2.T, b2.reshape(1, 2))


# speedup vs baseline: 16.5242x; 16.5242x over previous
import jax
import jax.numpy as jnp
from jax import lax
from jax.experimental import pallas as pl
from jax.experimental.pallas import tpu as pltpu
from jax.experimental.pallas import tpu_sc as plsc

_B = 16384
_V = 1000000
_H = _V // 2
_D = 64
_F = 128
_NC = 2
_NS = 16
_NW = _NC * _NS
_NIDX = 2 * _B
_PER_W = _NIDX // _NW
_RND = 2
_PER_RND = _PER_W // _RND
_NQ = _PER_RND // 128

_TBLK = 4096
_HP = 122 * _TBLK
_TROWS = 123 * _TBLK
_TGRID = _TROWS // _TBLK

_mesh = plsc.VectorSubcoreMesh(
    core_axis_name="c", subcore_axis_name="s",
    num_cores=_NC, num_subcores=_NS,
)


def _pack_body(lo_ref, hi_ref, out_ref):
    out_ref[:, :_D] = jnp.swapaxes(lo_ref[...], 0, 1)
    out_ref[:, _D:] = jnp.swapaxes(hi_ref[...], 0, 1)


def _pack(embt):
    return pl.pallas_call(
        _pack_body,
        grid=(_TGRID,),
        in_specs=[
            pl.BlockSpec((_D, _TBLK), lambda i: (0, i)),
            pl.BlockSpec((_D, _TBLK), lambda i: (0, i + _HP // _TBLK)),
        ],
        out_specs=pl.BlockSpec((_TBLK, 2 * _D), lambda i: (i, 0)),
        out_shape=jax.ShapeDtypeStruct((_TROWS, 2 * _D), jnp.float32),
    )(embt, embt)


def _gather_body(tbl_hbm, idx_hbm, out_hbm, idx_v, rows_v, sem):
    wid = lax.axis_index("s") * _NC + lax.axis_index("c")
    for r in range(_RND):
        pltpu.sync_copy(idx_hbm.at[wid, r], idx_v)
        cps = [
            pltpu.async_copy(
                tbl_hbm.at[idx_v.at[q]],
                rows_v.at[pl.ds(q * 128, 128)],
                sem,
            )
            for q in range(_NQ)
        ]
        for cp in cps:
            cp.wait()
        base = wid * _PER_W + r * _PER_RND
        pltpu.sync_copy(rows_v, out_hbm.at[pl.ds(base, _PER_RND)])


_gather = pl.kernel(
    _gather_body,
    out_type=jax.ShapeDtypeStruct((_NIDX, 2 * _D), jnp.float32),
    mesh=_mesh,
    scratch_types=[
        pltpu.VMEM((_NQ, 128), jnp.int32),
        pltpu.VMEM((_PER_RND, 2 * _D), jnp.float32),
        pltpu.SemaphoreType.DMA,
    ],
)

_BT = 2048


def _mlp_body(top_ref, bot_ref, p0_ref, p1_ref, w1at_ref, w1bt_ref, b1_ref,
              w2t_ref, b2_ref, out_ref):
    top = top_ref[...]
    bot = bot_ref[...]
    e0 = jnp.where(p0_ref[...], top[:, _D:], top[:, :_D])
    e1 = jnp.where(p1_ref[...], bot[:, _D:], bot[:, :_D])
    cond = (
        jnp.dot(e0, w1at_ref[...], preferred_element_type=jnp.float32)
        + jnp.dot(e1, w1bt_ref[...], preferred_element_type=jnp.float32)
        + b1_ref[...]
    )
    h = jnp.tanh(cond)
    out_ref[...] = (
        jnp.dot(h, w2t_ref[...], preferred_element_type=jnp.float32)
        + b2_ref[...]
    )


def _mlp(g, p0, p1, w1at, w1bt, b1r, w2t, b2r):
    return pl.pallas_call(
        _mlp_body,
        grid=(_B // _BT,),
        in_specs=[
            pl.BlockSpec((_BT, 2 * _D), lambda n: (n, 0)),
            pl.BlockSpec((_BT, 2 * _D), lambda n: (n + _B // _BT, 0)),
            pl.BlockSpec((_BT, 1), lambda n: (n, 0)),
            pl.BlockSpec((_BT, 1), lambda n: (n, 0)),
            pl.BlockSpec((_D, _F), lambda n: (0, 0)),
            pl.BlockSpec((_D, _F), lambda n: (0, 0)),
            pl.BlockSpec((1, _F), lambda n: (0, 0)),
            pl.BlockSpec((_F, 2), lambda n: (0, 0)),
            pl.BlockSpec((1, 2), lambda n: (0, 0)),
        ],
        out_specs=pl.BlockSpec((_BT, 2), lambda n: (n, 0)),
        out_shape=jax.ShapeDtypeStruct((_B, 2), jnp.float32),
    )(g, g, p0, p1, w1at, w1bt, b1r, w2t, b2r)


def kernel(x, x_mask, ents, batch_spos, batch_tpos, batch_sent_chars, emb, W1, b1, W2, b2):
    tbl = _pack(emb.T)

    ents32 = ents.astype(jnp.int32)
    half = ents32 >= _HP
    row = jnp.where(half, ents32 - _HP, ents32)
    idx = row.T.reshape(_NW, _RND, _NQ, 128)

    g = _gather(tbl, idx)

    return _mlp(
        g,
        half[:, 0].reshape(_B, 1),
        half[:, 1].reshape(_B, 1),
        W1[:, :_D].T, W1[:, _D:].T,
        b1.reshape(1, _F),
        W2.T,
        b2.reshape(1, 2),
    )
